# baseline (speedup 1.0000x reference)
import functools

import jax
import jax.numpy as jnp
from jax import lax
from jax.experimental import pallas as pl
from jax.experimental.pallas import tpu as pltpu
from jax.experimental.pallas import tpu_sc as plsc

B = 16384
K = 5
D = 32
N = B * (1 + K)
NC = 2
NS = 16
NW = NC * NS
CHUNK = 128
N_CH = N // (NW * CHUNK)
ROWS = N // CHUNK

TWO_PI = 6.283185307179586
_C0 = 0.9999994436787928
_C1 = -0.49999558165605595
_C2 = 0.04166103279014615
_C3 = -0.0013862747315839738
_C4 = 2.4253192495694853e-05
_C5 = -2.2193949944515623e-07


def _sc_body(u_hbm, v_hbm, aw_hbm, pw_hbm, av_hbm, pv_hbm, out_hbm,
             idx_u, idx_v, au, pu, av, pv, sc_out, sem):
    wid = lax.axis_index("s") * NC + lax.axis_index("c")
    base = wid * N_CH
    pltpu.sync_copy(u_hbm.at[pl.ds(base, N_CH)], idx_u)
    pltpu.sync_copy(v_hbm.at[pl.ds(base, N_CH)], idx_v)

    def chunk(j, carry):
        c1 = pltpu.async_copy(aw_hbm.at[idx_u.at[j]], au, sem)
        c2 = pltpu.async_copy(pw_hbm.at[idx_u.at[j]], pu, sem)
        c3 = pltpu.async_copy(av_hbm.at[idx_v.at[j]], av, sem)
        c4 = pltpu.async_copy(pv_hbm.at[idx_v.at[j]], pv, sem)
        c1.wait(); c2.wait(); c3.wait(); c4.wait()

        def grp(g, carry2):
            rows = g * 16 + lax.iota(jnp.int32, 16)

            def dim_step(dd, acc):
                cols = jnp.full((16,), dd, jnp.int32)
                a1 = plsc.load_gather(au, [rows, cols])
                p1 = plsc.load_gather(pu, [rows, cols])
                a2 = plsc.load_gather(av, [rows, cols])
                p2 = plsc.load_gather(pv, [rows, cols])
                ax = jnp.abs(p1 - p2)
                y = jnp.minimum(ax, TWO_PI - ax)
                uu = y * y
                cosv = _C0 + uu * (_C1 + uu * (_C2 + uu * (
                    _C3 + uu * (_C4 + uu * _C5))))
                return acc + a1 * a2 * cosv

            acc = lax.fori_loop(0, D, dim_step, jnp.zeros((16,), jnp.float32))
            sc_out[j, pl.ds(g * 16, 16)] = acc
            return carry2

        lax.fori_loop(0, CHUNK // 16, grp, 0)
        return carry

    lax.fori_loop(0, N_CH, chunk, 0)
    pltpu.sync_copy(sc_out, out_hbm.at[pl.ds(base, N_CH)])


_sc_scores = functools.partial(
    pl.kernel,
    out_type=jax.ShapeDtypeStruct((ROWS, CHUNK), jnp.float32),
    mesh=plsc.VectorSubcoreMesh(core_axis_name="c", subcore_axis_name="s",
                                num_cores=NC, num_subcores=NS),
    scratch_types=[
        pltpu.VMEM((N_CH, CHUNK), jnp.int32),
        pltpu.VMEM((N_CH, CHUNK), jnp.int32),
        pltpu.VMEM((CHUNK, D), jnp.float32),
        pltpu.VMEM((CHUNK, D), jnp.float32),
        pltpu.VMEM((CHUNK, D), jnp.float32),
        pltpu.VMEM((CHUNK, D), jnp.float32),
        pltpu.VMEM((N_CH, CHUNK), jnp.float32),
        pltpu.SemaphoreType.DMA,
    ],
    compiler_params=pltpu.CompilerParams(needs_layout_passes=False,
                                         use_tc_tiling_on_sc=False),
)(_sc_body)


def _tc_loss_body(s_ref, o_ref):
    s = s_ref[...]
    row = lax.broadcasted_iota(jnp.int32, (ROWS, CHUNK), 0)
    z = jnp.where(row < B // CHUNK, s, -s)
    ls = jnp.minimum(z, 0.0) - jnp.log1p(jnp.exp(-jnp.abs(z)))
    o_ref[...] = jnp.reshape(-jnp.sum(ls) * (1.0 / B), (1, 1))


_tc_loss = pl.pallas_call(
    _tc_loss_body,
    out_shape=jax.ShapeDtypeStruct((1, 1), jnp.float32),
)


def kernel(pos_u, pos_v, neg_u, neg_v,
           word_amplitude_w, word_phase_w, word_amplitude_v, word_phase_v):
    u_all = jnp.concatenate(
        [pos_u.astype(jnp.int32), neg_u.reshape(-1).astype(jnp.int32)]
    ).reshape(ROWS, CHUNK)
    v_all = jnp.concatenate(
        [pos_v.astype(jnp.int32), neg_v.reshape(-1).astype(jnp.int32)]
    ).reshape(ROWS, CHUNK)
    scores = _sc_scores(u_all, v_all, word_amplitude_w, word_phase_w,
                        word_amplitude_v, word_phase_v)
    return _tc_loss(scores)[0, 0]


# speedup vs baseline: 1.0158x; 1.0158x over previous
import functools

import jax
import jax.numpy as jnp
from jax import lax
from jax.experimental import pallas as pl
from jax.experimental.pallas import tpu as pltpu
from jax.experimental.pallas import tpu_sc as plsc

B = 16384
K = 5
D = 32
N = B * (1 + K)
NC = 2
NS = 16
NW = NC * NS
CHUNK = 128
N_CH = N // (NW * CHUNK)
ROWS = N // CHUNK

TWO_PI = 6.283185307179586
_C0 = 0.9999994436787928
_C1 = -0.49999558165605595
_C2 = 0.04166103279014615
_C3 = -0.0013862747315839738
_C4 = 2.4253192495694853e-05
_C5 = -2.2193949944515623e-07


def _sc_body(u_hbm, v_hbm, aw_hbm, pw_hbm, av_hbm, pv_hbm, out_hbm,
             idx_u, idx_v, bufs0, bufs1, sc_out, sem0, sem1):
    wid = lax.axis_index("s") * NC + lax.axis_index("c")
    base = wid * N_CH
    pltpu.sync_copy(u_hbm.at[pl.ds(base, N_CH)], idx_u)
    pltpu.sync_copy(v_hbm.at[pl.ds(base, N_CH)], idx_v)

    def dmas(j, bufs, sem):
        au, pu, av, pv = bufs
        return (pltpu.make_async_copy(aw_hbm.at[idx_u.at[j]], au, sem),
                pltpu.make_async_copy(pw_hbm.at[idx_u.at[j]], pu, sem),
                pltpu.make_async_copy(av_hbm.at[idx_v.at[j]], av, sem),
                pltpu.make_async_copy(pv_hbm.at[idx_v.at[j]], pv, sem))

    def fire(j, bufs, sem):
        for c in dmas(j, bufs, sem):
            c.start()

    def drain(j, bufs, sem):
        for c in dmas(j, bufs, sem):
            c.wait()

    def compute(j, bufs):
        au, pu, av, pv = bufs

        def grp(g, carry2):
            rows = g * 16 + lax.iota(jnp.int32, 16)

            def dim_step(dd, acc):
                cols = jnp.full((16,), dd, jnp.int32)
                a1 = plsc.load_gather(au, [rows, cols])
                p1 = plsc.load_gather(pu, [rows, cols])
                a2 = plsc.load_gather(av, [rows, cols])
                p2 = plsc.load_gather(pv, [rows, cols])
                ax = jnp.abs(p1 - p2)
                y = jnp.minimum(ax, TWO_PI - ax)
                uu = y * y
                cosv = _C0 + uu * (_C1 + uu * (_C2 + uu * (
                    _C3 + uu * (_C4 + uu * _C5))))
                return acc + a1 * a2 * cosv

            acc = lax.fori_loop(0, D, dim_step, jnp.zeros((16,), jnp.float32))
            sc_out[j, pl.ds(g * 16, 16)] = acc
            return carry2

        lax.fori_loop(0, CHUNK // 16, grp, 0)

    b0 = tuple(bufs0)
    b1 = tuple(bufs1)
    fire(0, b0, sem0)

    def pair_step(i, carry):
        j0 = 2 * i
        j1 = j0 + 1
        fire(j1, b1, sem1)
        drain(j0, b0, sem0)
        compute(j0, b0)

        @pl.when(j0 + 2 < N_CH)
        def _():
            fire(j0 + 2, b0, sem0)

        drain(j1, b1, sem1)
        compute(j1, b1)
        return carry

    lax.fori_loop(0, N_CH // 2, pair_step, 0)
    pltpu.sync_copy(sc_out, out_hbm.at[pl.ds(base, N_CH)])


_sc_scores = functools.partial(
    pl.kernel,
    out_type=jax.ShapeDtypeStruct((ROWS, CHUNK), jnp.float32),
    mesh=plsc.VectorSubcoreMesh(core_axis_name="c", subcore_axis_name="s",
                                num_cores=NC, num_subcores=NS),
    scratch_types=[
        pltpu.VMEM((N_CH, CHUNK), jnp.int32),
        pltpu.VMEM((N_CH, CHUNK), jnp.int32),
        tuple(pltpu.VMEM((CHUNK, D), jnp.float32) for _ in range(4)),
        tuple(pltpu.VMEM((CHUNK, D), jnp.float32) for _ in range(4)),
        pltpu.VMEM((N_CH, CHUNK), jnp.float32),
        pltpu.SemaphoreType.DMA,
        pltpu.SemaphoreType.DMA,
    ],
    compiler_params=pltpu.CompilerParams(needs_layout_passes=False,
                                         use_tc_tiling_on_sc=False),
)(_sc_body)


def _tc_loss_body(s_ref, o_ref):
    s = s_ref[...]
    row = lax.broadcasted_iota(jnp.int32, (ROWS, CHUNK), 0)
    z = jnp.where(row < B // CHUNK, s, -s)
    ls = jnp.minimum(z, 0.0) - jnp.log1p(jnp.exp(-jnp.abs(z)))
    o_ref[...] = jnp.reshape(-jnp.sum(ls) * (1.0 / B), (1, 1))


_tc_loss = pl.pallas_call(
    _tc_loss_body,
    out_shape=jax.ShapeDtypeStruct((1, 1), jnp.float32),
)


def kernel(pos_u, pos_v, neg_u, neg_v,
           word_amplitude_w, word_phase_w, word_amplitude_v, word_phase_v):
    u_all = jnp.concatenate(
        [pos_u.astype(jnp.int32), neg_u.reshape(-1).astype(jnp.int32)]
    ).reshape(ROWS, CHUNK)
    v_all = jnp.concatenate(
        [pos_v.astype(jnp.int32), neg_v.reshape(-1).astype(jnp.int32)]
    ).reshape(ROWS, CHUNK)
    scores = _sc_scores(u_all, v_all, word_amplitude_w, word_phase_w,
                        word_amplitude_v, word_phase_v)
    return _tc_loss(scores)[0, 0]


# speedup vs baseline: 1.2587x; 1.2392x over previous
import functools

import jax
import jax.numpy as jnp
from jax import lax
from jax.experimental import pallas as pl
from jax.experimental.pallas import tpu as pltpu
from jax.experimental.pallas import tpu_sc as plsc

B = 16384
K = 5
D = 32
N = B * (1 + K)
NC = 2
NS = 16
NW = NC * NS
CHUNK = 128
N_CH = N // (NW * CHUNK)
ROWS = N // CHUNK

V = 1000000
TBLK = 2048
TGRID = (V + TBLK - 1) // TBLK
PACKED_ROWS = TGRID * (TBLK // 4)

TWO_PI = 6.283185307179586
_C0 = 0.9999994436787928
_C1 = -0.49999558165605595
_C2 = 0.04166103279014615
_C3 = -0.0013862747315839738
_C4 = 2.4253192495694853e-05
_C5 = -2.2193949944515623e-07


def _tr_body(a_ref, p_ref, oa_ref, op_ref):
    for ref, oref in ((a_ref, oa_ref), (p_ref, op_ref)):
        x = ref[...]
        oref[...] = jnp.concatenate(
            [jnp.swapaxes(x[:, 512 * c:512 * (c + 1)], 0, 1) for c in range(4)],
            axis=1)


_tc_transpose2 = pl.pallas_call(
    _tr_body,
    grid=(TGRID,),
    in_specs=[pl.BlockSpec((D, TBLK), lambda g: (0, g)),
              pl.BlockSpec((D, TBLK), lambda g: (0, g))],
    out_specs=[pl.BlockSpec((512, 128), lambda g: (g, 0)),
               pl.BlockSpec((512, 128), lambda g: (g, 0))],
    out_shape=[jax.ShapeDtypeStruct((PACKED_ROWS, 128), jnp.float32),
               jax.ShapeDtypeStruct((PACKED_ROWS, 128), jnp.float32)],
)


def _sc_body(u_hbm, ou_hbm, v_hbm, ov_hbm, aw_hbm, pw_hbm, av_hbm, pv_hbm,
             out_hbm, idx_u, off_u, idx_v, off_v, au, pu, av, pv, sc_out, sem):
    wid = lax.axis_index("s") * NC + lax.axis_index("c")
    base = wid * N_CH
    pltpu.sync_copy(u_hbm.at[pl.ds(base, N_CH)], idx_u)
    pltpu.sync_copy(ou_hbm.at[pl.ds(base, N_CH)], off_u)
    pltpu.sync_copy(v_hbm.at[pl.ds(base, N_CH)], idx_v)
    pltpu.sync_copy(ov_hbm.at[pl.ds(base, N_CH)], off_v)

    def chunk(j, carry):
        c1 = pltpu.async_copy(aw_hbm.at[idx_u.at[j]], au, sem)
        c2 = pltpu.async_copy(pw_hbm.at[idx_u.at[j]], pu, sem)
        c3 = pltpu.async_copy(av_hbm.at[idx_v.at[j]], av, sem)
        c4 = pltpu.async_copy(pv_hbm.at[idx_v.at[j]], pv, sem)
        c1.wait(); c2.wait(); c3.wait(); c4.wait()

        def grp(g, carry2):
            rows = g * 16 + lax.iota(jnp.int32, 16)
            ou = off_u[j, pl.ds(g * 16, 16)]
            ov = off_v[j, pl.ds(g * 16, 16)]

            def dim_step(dd, acc):
                cu = ou + dd
                cv = ov + dd
                a1 = plsc.load_gather(au, [rows, cu])
                p1 = plsc.load_gather(pu, [rows, cu])
                a2 = plsc.load_gather(av, [rows, cv])
                p2 = plsc.load_gather(pv, [rows, cv])
                ax = jnp.abs(p1 - p2)
                y = jnp.minimum(ax, TWO_PI - ax)
                uu = y * y
                cosv = _C0 + uu * (_C1 + uu * (_C2 + uu * (
                    _C3 + uu * (_C4 + uu * _C5))))
                return acc + a1 * a2 * cosv

            acc = lax.fori_loop(0, D, dim_step, jnp.zeros((16,), jnp.float32))
            sc_out[j, pl.ds(g * 16, 16)] = acc
            return carry2

        lax.fori_loop(0, CHUNK // 16, grp, 0)
        return carry

    lax.fori_loop(0, N_CH, chunk, 0)
    pltpu.sync_copy(sc_out, out_hbm.at[pl.ds(base, N_CH)])


_sc_scores = functools.partial(
    pl.kernel,
    out_type=jax.ShapeDtypeStruct((ROWS, CHUNK), jnp.float32),
    mesh=plsc.VectorSubcoreMesh(core_axis_name="c", subcore_axis_name="s",
                                num_cores=NC, num_subcores=NS),
    scratch_types=[
        pltpu.VMEM((N_CH, CHUNK), jnp.int32),
        pltpu.VMEM((N_CH, CHUNK), jnp.int32),
        pltpu.VMEM((N_CH, CHUNK), jnp.int32),
        pltpu.VMEM((N_CH, CHUNK), jnp.int32),
        pltpu.VMEM((CHUNK, 128), jnp.float32),
        pltpu.VMEM((CHUNK, 128), jnp.float32),
        pltpu.VMEM((CHUNK, 128), jnp.float32),
        pltpu.VMEM((CHUNK, 128), jnp.float32),
        pltpu.VMEM((N_CH, CHUNK), jnp.float32),
        pltpu.SemaphoreType.DMA,
    ],
    compiler_params=pltpu.CompilerParams(needs_layout_passes=False),
)(_sc_body)


def _tc_loss_body(s_ref, o_ref):
    s = s_ref[...]
    row = lax.broadcasted_iota(jnp.int32, (ROWS, CHUNK), 0)
    z = jnp.where(row < B // CHUNK, s, -s)
    ls = jnp.minimum(z, 0.0) - jnp.log1p(jnp.exp(-jnp.abs(z)))
    o_ref[...] = jnp.reshape(-jnp.sum(ls) * (1.0 / B), (1, 1))


_tc_loss = pl.pallas_call(
    _tc_loss_body,
    out_shape=jax.ShapeDtypeStruct((1, 1), jnp.float32),
)


def kernel(pos_u, pos_v, neg_u, neg_v,
           word_amplitude_w, word_phase_w, word_amplitude_v, word_phase_v):
    u_all = jnp.concatenate(
        [pos_u.astype(jnp.int32), neg_u.reshape(-1).astype(jnp.int32)])
    v_all = jnp.concatenate(
        [pos_v.astype(jnp.int32), neg_v.reshape(-1).astype(jnp.int32)])
    u4 = ((u_all >> 11) * 512 + (u_all & 511)).reshape(ROWS, CHUNK)
    offu = (((u_all >> 9) & 3) * 32).reshape(ROWS, CHUNK)
    v4 = ((v_all >> 11) * 512 + (v_all & 511)).reshape(ROWS, CHUNK)
    offv = (((v_all >> 9) & 3) * 32).reshape(ROWS, CHUNK)
    rm_aw, rm_pw = _tc_transpose2(word_amplitude_w.T, word_phase_w.T)
    rm_av, rm_pv = _tc_transpose2(word_amplitude_v.T, word_phase_v.T)
    scores = _sc_scores(u4, offu, v4, offv, rm_aw, rm_pw, rm_av, rm_pv)
    return _tc_loss(scores)[0, 0]


# speedup vs baseline: 1.2594x; 1.0005x over previous
import functools

import jax
import jax.numpy as jnp
from jax import lax
from jax.experimental import pallas as pl
from jax.experimental.pallas import tpu as pltpu
from jax.experimental.pallas import tpu_sc as plsc

B = 16384
K = 5
D = 32
N = B * (1 + K)
NC = 2
NS = 16
NW = NC * NS
CHUNK = 128
N_CH = N // (NW * CHUNK)
ROWS = N // CHUNK

V = 1000000
TBLK = 2048
TGRID = (V + TBLK - 1) // TBLK
PACKED_ROWS = TGRID * (TBLK // 4)

TWO_PI = 6.283185307179586
_C0 = 0.9999994436787928
_C1 = -0.49999558165605595
_C2 = 0.04166103279014615
_C3 = -0.0013862747315839738
_C4 = 2.4253192495694853e-05
_C5 = -2.2193949944515623e-07


def _tr_body(a_ref, p_ref, oa_ref, op_ref):
    for ref, oref in ((a_ref, oa_ref), (p_ref, op_ref)):
        x = ref[...]
        oref[...] = jnp.concatenate(
            [jnp.swapaxes(x[:, 512 * c:512 * (c + 1)], 0, 1) for c in range(4)],
            axis=1)


_tc_transpose2 = pl.pallas_call(
    _tr_body,
    grid=(TGRID,),
    in_specs=[pl.BlockSpec((D, TBLK), lambda g: (0, g)),
              pl.BlockSpec((D, TBLK), lambda g: (0, g))],
    out_specs=[pl.BlockSpec((512, 128), lambda g: (g, 0)),
               pl.BlockSpec((512, 128), lambda g: (g, 0))],
    out_shape=[jax.ShapeDtypeStruct((PACKED_ROWS, 128), jnp.float32),
               jax.ShapeDtypeStruct((PACKED_ROWS, 128), jnp.float32)],
)


def _sc_body(u_hbm, ou_hbm, v_hbm, ov_hbm, aw_hbm, pw_hbm, av_hbm, pv_hbm,
             out_hbm, idx_u, off_u, idx_v, off_v, au, pu, av, pv, sc_out, sem):
    wid = lax.axis_index("s") * NC + lax.axis_index("c")
    base = wid * N_CH
    pltpu.sync_copy(u_hbm.at[pl.ds(base, N_CH)], idx_u)
    pltpu.sync_copy(ou_hbm.at[pl.ds(base, N_CH)], off_u)
    pltpu.sync_copy(v_hbm.at[pl.ds(base, N_CH)], idx_v)
    pltpu.sync_copy(ov_hbm.at[pl.ds(base, N_CH)], off_v)

    def chunk(j, carry):
        c1 = pltpu.async_copy(aw_hbm.at[idx_u.at[j]], au, sem)
        c2 = pltpu.async_copy(pw_hbm.at[idx_u.at[j]], pu, sem)
        c3 = pltpu.async_copy(av_hbm.at[idx_v.at[j]], av, sem)
        c4 = pltpu.async_copy(pv_hbm.at[idx_v.at[j]], pv, sem)
        c1.wait(); c2.wait(); c3.wait(); c4.wait()

        def grp(g, carry2):
            rows = g * 16 + lax.iota(jnp.int32, 16)
            ou = off_u[j, pl.ds(g * 16, 16)]
            ov = off_v[j, pl.ds(g * 16, 16)]

            def dim_step(q, acc):
                for k in range(8):
                    dd = q * 8 + k
                    cu = ou + dd
                    cv = ov + dd
                    a1 = plsc.load_gather(au, [rows, cu])
                    p1 = plsc.load_gather(pu, [rows, cu])
                    a2 = plsc.load_gather(av, [rows, cv])
                    p2 = plsc.load_gather(pv, [rows, cv])
                    ax = jnp.abs(p1 - p2)
                    y = jnp.minimum(ax, TWO_PI - ax)
                    uu = y * y
                    cosv = _C0 + uu * (_C1 + uu * (_C2 + uu * (
                        _C3 + uu * (_C4 + uu * _C5))))
                    acc = acc + a1 * a2 * cosv
                return acc

            acc = lax.fori_loop(0, D // 8, dim_step,
                                jnp.zeros((16,), jnp.float32))
            sc_out[j, pl.ds(g * 16, 16)] = acc
            return carry2

        lax.fori_loop(0, CHUNK // 16, grp, 0)
        return carry

    lax.fori_loop(0, N_CH, chunk, 0)
    pltpu.sync_copy(sc_out, out_hbm.at[pl.ds(base, N_CH)])


_sc_scores = functools.partial(
    pl.kernel,
    out_type=jax.ShapeDtypeStruct((ROWS, CHUNK), jnp.float32),
    mesh=plsc.VectorSubcoreMesh(core_axis_name="c", subcore_axis_name="s",
                                num_cores=NC, num_subcores=NS),
    scratch_types=[
        pltpu.VMEM((N_CH, CHUNK), jnp.int32),
        pltpu.VMEM((N_CH, CHUNK), jnp.int32),
        pltpu.VMEM((N_CH, CHUNK), jnp.int32),
        pltpu.VMEM((N_CH, CHUNK), jnp.int32),
        pltpu.VMEM((CHUNK, 128), jnp.float32),
        pltpu.VMEM((CHUNK, 128), jnp.float32),
        pltpu.VMEM((CHUNK, 128), jnp.float32),
        pltpu.VMEM((CHUNK, 128), jnp.float32),
        pltpu.VMEM((N_CH, CHUNK), jnp.float32),
        pltpu.SemaphoreType.DMA,
    ],
    compiler_params=pltpu.CompilerParams(needs_layout_passes=False),
)(_sc_body)


def _tc_loss_body(s_ref, o_ref):
    s = s_ref[...]
    row = lax.broadcasted_iota(jnp.int32, (ROWS, CHUNK), 0)
    z = jnp.where(row < B // CHUNK, s, -s)
    ls = jnp.minimum(z, 0.0) - jnp.log1p(jnp.exp(-jnp.abs(z)))
    o_ref[...] = jnp.reshape(-jnp.sum(ls) * (1.0 / B), (1, 1))


_tc_loss = pl.pallas_call(
    _tc_loss_body,
    out_shape=jax.ShapeDtypeStruct((1, 1), jnp.float32),
)


def kernel(pos_u, pos_v, neg_u, neg_v,
           word_amplitude_w, word_phase_w, word_amplitude_v, word_phase_v):
    u_all = jnp.concatenate(
        [pos_u.astype(jnp.int32), neg_u.reshape(-1).astype(jnp.int32)])
    v_all = jnp.concatenate(
        [pos_v.astype(jnp.int32), neg_v.reshape(-1).astype(jnp.int32)])
    u4 = ((u_all >> 11) * 512 + (u_all & 511)).reshape(ROWS, CHUNK)
    offu = (((u_all >> 9) & 3) * 32).reshape(ROWS, CHUNK)
    v4 = ((v_all >> 11) * 512 + (v_all & 511)).reshape(ROWS, CHUNK)
    offv = (((v_all >> 9) & 3) * 32).reshape(ROWS, CHUNK)
    rm_aw, rm_pw = _tc_transpose2(word_amplitude_w.T, word_phase_w.T)
    rm_av, rm_pv = _tc_transpose2(word_amplitude_v.T, word_phase_v.T)
    scores = _sc_scores(u4, offu, v4, offv, rm_aw, rm_pw, rm_av, rm_pv)
    return _tc_loss(scores)[0, 0]


# speedup vs baseline: 1.8520x; 1.4706x over previous
import functools

import jax
import jax.numpy as jnp
from jax import lax
from jax.experimental import pallas as pl
from jax.experimental.pallas import tpu as pltpu
from jax.experimental.pallas import tpu_sc as plsc

B = 16384
K = 5
D = 32
N = B * (1 + K)
NC = 2
NS = 16
NW = NC * NS
CHUNK = 128
N_CH = N // (NW * CHUNK)
ROWS = N // CHUNK

V = 1000000
TBLK = 2048
TGRID = (V + TBLK - 1) // TBLK
PACKED_ROWS = TGRID * (TBLK // 4)

TWO_PI = 6.283185307179586
_C0 = 0.9999994436787928
_C1 = -0.49999558165605595
_C2 = 0.04166103279014615
_C3 = -0.0013862747315839738
_C4 = 2.4253192495694853e-05
_C5 = -2.2193949944515623e-07


def _bf16_rne_bits(x):
    u = lax.bitcast_convert_type(x, jnp.int32)
    return u + jnp.int32(0x7FFF) + (lax.shift_right_logical(u, 16) & 1)


def _tr_body(aw_ref, pw_ref, av_ref, pv_ref, ou_ref, ov_ref):
    for aref, pref, oref in ((aw_ref, pw_ref, ou_ref),
                             (av_ref, pv_ref, ov_ref)):
        a = aref[...]
        p = pref[...]
        pieces = []
        for c in range(4):
            sl = slice(512 * c, 512 * (c + 1))
            ra = _bf16_rne_bits(jnp.swapaxes(a[:, sl], 0, 1))
            rp = _bf16_rne_bits(jnp.swapaxes(p[:, sl], 0, 1))
            pieces.append((rp & jnp.int32(-65536))
                          | lax.shift_right_logical(ra, 16))
        oref[...] = jnp.concatenate(pieces, axis=1)


_tc_repack = pl.pallas_call(
    _tr_body,
    grid=(TGRID,),
    in_specs=[pl.BlockSpec((D, TBLK), lambda g: (0, g)) for _ in range(4)],
    out_specs=[pl.BlockSpec((512, 128), lambda g: (g, 0)) for _ in range(2)],
    out_shape=[jax.ShapeDtypeStruct((PACKED_ROWS, 128), jnp.int32)
               for _ in range(2)],
)


def _sc_body(u_hbm, ou_hbm, v_hbm, ov_hbm, pku_hbm, pkv_hbm,
             out_hbm, idx_u, off_u, idx_v, off_v, bu, bv, sc_out, sem):
    wid = lax.axis_index("s") * NC + lax.axis_index("c")
    base = wid * N_CH
    pltpu.sync_copy(u_hbm.at[pl.ds(base, N_CH)], idx_u)
    pltpu.sync_copy(ou_hbm.at[pl.ds(base, N_CH)], off_u)
    pltpu.sync_copy(v_hbm.at[pl.ds(base, N_CH)], idx_v)
    pltpu.sync_copy(ov_hbm.at[pl.ds(base, N_CH)], off_v)

    def chunk(j, carry):
        c1 = pltpu.async_copy(pku_hbm.at[idx_u.at[j]], bu, sem)
        c2 = pltpu.async_copy(pkv_hbm.at[idx_v.at[j]], bv, sem)
        c1.wait(); c2.wait()

        def grp(g, carry2):
            rows = g * 16 + lax.iota(jnp.int32, 16)
            ou = off_u[j, pl.ds(g * 16, 16)]
            ov = off_v[j, pl.ds(g * 16, 16)]

            def dim_step(q, acc):
                for k in range(8):
                    dd = q * 8 + k
                    xu = plsc.load_gather(bu, [rows, ou + dd])
                    xv = plsc.load_gather(bv, [rows, ov + dd])
                    a1, p1 = plsc.unpack(plsc.bitcast(xu, jnp.bfloat16),
                                         format=plsc.PackFormat.INTERLEAVED)
                    a2, p2 = plsc.unpack(plsc.bitcast(xv, jnp.bfloat16),
                                         format=plsc.PackFormat.INTERLEAVED)
                    ax = jnp.abs(p1 - p2)
                    y = jnp.minimum(ax, TWO_PI - ax)
                    uu = y * y
                    cosv = _C0 + uu * (_C1 + uu * (_C2 + uu * (
                        _C3 + uu * (_C4 + uu * _C5))))
                    acc = acc + a1 * a2 * cosv
                return acc

            acc = lax.fori_loop(0, D // 8, dim_step,
                                jnp.zeros((16,), jnp.float32))
            sc_out[j, pl.ds(g * 16, 16)] = acc
            return carry2

        lax.fori_loop(0, CHUNK // 16, grp, 0)
        return carry

    lax.fori_loop(0, N_CH, chunk, 0)
    pltpu.sync_copy(sc_out, out_hbm.at[pl.ds(base, N_CH)])


_sc_scores = functools.partial(
    pl.kernel,
    out_type=jax.ShapeDtypeStruct((ROWS, CHUNK), jnp.float32),
    mesh=plsc.VectorSubcoreMesh(core_axis_name="c", subcore_axis_name="s",
                                num_cores=NC, num_subcores=NS),
    scratch_types=[
        pltpu.VMEM((N_CH, CHUNK), jnp.int32),
        pltpu.VMEM((N_CH, CHUNK), jnp.int32),
        pltpu.VMEM((N_CH, CHUNK), jnp.int32),
        pltpu.VMEM((N_CH, CHUNK), jnp.int32),
        pltpu.VMEM((CHUNK, 128), jnp.int32),
        pltpu.VMEM((CHUNK, 128), jnp.int32),
        pltpu.VMEM((N_CH, CHUNK), jnp.float32),
        pltpu.SemaphoreType.DMA,
    ],
    compiler_params=pltpu.CompilerParams(needs_layout_passes=False),
)(_sc_body)


def _tc_loss_body(s_ref, o_ref):
    s = s_ref[...]
    row = lax.broadcasted_iota(jnp.int32, (ROWS, CHUNK), 0)
    z = jnp.where(row < B // CHUNK, s, -s)
    ls = jnp.minimum(z, 0.0) - jnp.log1p(jnp.exp(-jnp.abs(z)))
    o_ref[...] = jnp.reshape(-jnp.sum(ls) * (1.0 / B), (1, 1))


_tc_loss = pl.pallas_call(
    _tc_loss_body,
    out_shape=jax.ShapeDtypeStruct((1, 1), jnp.float32),
)


def kernel(pos_u, pos_v, neg_u, neg_v,
           word_amplitude_w, word_phase_w, word_amplitude_v, word_phase_v):
    u_all = jnp.concatenate(
        [pos_u.astype(jnp.int32), neg_u.reshape(-1).astype(jnp.int32)])
    v_all = jnp.concatenate(
        [pos_v.astype(jnp.int32), neg_v.reshape(-1).astype(jnp.int32)])
    u4 = ((u_all >> 11) * 512 + (u_all & 511)).reshape(ROWS, CHUNK)
    offu = (((u_all >> 9) & 3) * 32).reshape(ROWS, CHUNK)
    v4 = ((v_all >> 11) * 512 + (v_all & 511)).reshape(ROWS, CHUNK)
    offv = (((v_all >> 9) & 3) * 32).reshape(ROWS, CHUNK)
    pku, pkv = _tc_repack(word_amplitude_w.T, word_phase_w.T,
                          word_amplitude_v.T, word_phase_v.T)
    scores = _sc_scores(u4, offu, v4, offv, pku, pkv)
    return _tc_loss(scores)[0, 0]


# speedup vs baseline: 1.9977x; 1.0787x over previous
import functools

import jax
import jax.numpy as jnp
from jax import lax
from jax.experimental import pallas as pl
from jax.experimental.pallas import tpu as pltpu
from jax.experimental.pallas import tpu_sc as plsc

B = 16384
K = 5
D = 32
N = B * (1 + K)
NC = 2
NS = 16
NW = NC * NS
CHUNK = 128
N_CH = N // (NW * CHUNK)
ROWS = N // CHUNK

V = 1000000
TBLK = 4096
TGRID = (V + TBLK - 1) // TBLK
PACKED_ROWS = TGRID * (TBLK // 4)

TWO_PI = 6.283185307179586
_C0 = 0.9999994436787928
_C1 = -0.49999558165605595
_C2 = 0.04166103279014615
_C3 = -0.0013862747315839738
_C4 = 2.4253192495694853e-05
_C5 = -2.2193949944515623e-07


def _bf16_rne_bits(x):
    u = lax.bitcast_convert_type(x, jnp.int32)
    return u + jnp.int32(0x7FFF) + (lax.shift_right_logical(u, 16) & 1)


def _tr_body(aw_ref, pw_ref, av_ref, pv_ref, ou_ref, ov_ref):
    for aref, pref, oref in ((aw_ref, pw_ref, ou_ref),
                             (av_ref, pv_ref, ov_ref)):
        a = aref[...]
        p = pref[...]
        pieces = []
        for c in range(TBLK // 512):
            sl = slice(512 * c, 512 * (c + 1))
            ra = _bf16_rne_bits(jnp.swapaxes(a[:, sl], 0, 1))
            rp = _bf16_rne_bits(jnp.swapaxes(p[:, sl], 0, 1))
            pieces.append((rp & jnp.int32(-65536))
                          | lax.shift_right_logical(ra, 16))
        oref[...] = jnp.concatenate(
            [jnp.concatenate(pieces[4 * s:4 * s + 4], axis=1)
             for s in range(TBLK // 2048)], axis=0)


_tc_repack = pl.pallas_call(
    _tr_body,
    grid=(TGRID,),
    in_specs=[pl.BlockSpec((D, TBLK), lambda g: (0, g)) for _ in range(4)],
    out_specs=[pl.BlockSpec((TBLK // 4, 128), lambda g: (g, 0)) for _ in range(2)],
    out_shape=[jax.ShapeDtypeStruct((PACKED_ROWS, 128), jnp.int32)
               for _ in range(2)],
)


def _sc_body(u_hbm, ou_hbm, v_hbm, ov_hbm, pku_hbm, pkv_hbm,
             out_hbm, idx_u, off_u, idx_v, off_v,
             bu0, bv0, bu1, bv1, sc_out, sem0, sem1):
    wid = lax.axis_index("s") * NC + lax.axis_index("c")
    base = wid * N_CH
    pltpu.sync_copy(u_hbm.at[pl.ds(base, N_CH)], idx_u)
    pltpu.sync_copy(ou_hbm.at[pl.ds(base, N_CH)], off_u)
    pltpu.sync_copy(v_hbm.at[pl.ds(base, N_CH)], idx_v)
    pltpu.sync_copy(ov_hbm.at[pl.ds(base, N_CH)], off_v)

    def dmas(j, bufs, sem):
        bu, bv = bufs
        return (pltpu.make_async_copy(pku_hbm.at[idx_u.at[j]], bu, sem),
                pltpu.make_async_copy(pkv_hbm.at[idx_v.at[j]], bv, sem))

    def fire(j, bufs, sem):
        for c in dmas(j, bufs, sem):
            c.start()

    def drain(j, bufs, sem):
        for c in dmas(j, bufs, sem):
            c.wait()

    def compute(j, bufs):
        bu, bv = bufs

        def grp(g, carry2):
            rows = g * 16 + lax.iota(jnp.int32, 16)
            ou = off_u[j, pl.ds(g * 16, 16)]
            ov = off_v[j, pl.ds(g * 16, 16)]

            def dim_step(q, acc):
                for k in range(8):
                    dd = q * 8 + k
                    xu = plsc.load_gather(bu, [rows, ou + dd])
                    xv = plsc.load_gather(bv, [rows, ov + dd])
                    a1, p1 = plsc.unpack(plsc.bitcast(xu, jnp.bfloat16),
                                         format=plsc.PackFormat.INTERLEAVED)
                    a2, p2 = plsc.unpack(plsc.bitcast(xv, jnp.bfloat16),
                                         format=plsc.PackFormat.INTERLEAVED)
                    ax = jnp.abs(p1 - p2)
                    y = jnp.minimum(ax, TWO_PI - ax)
                    uu = y * y
                    cosv = _C0 + uu * (_C1 + uu * (_C2 + uu * (
                        _C3 + uu * (_C4 + uu * _C5))))
                    acc = acc + a1 * a2 * cosv
                return acc

            acc = lax.fori_loop(0, D // 8, dim_step,
                                jnp.zeros((16,), jnp.float32))
            sc_out[j, pl.ds(g * 16, 16)] = acc
            return carry2

        lax.fori_loop(0, CHUNK // 16, grp, 0)

    b0 = (bu0, bv0)
    b1 = (bu1, bv1)
    fire(0, b0, sem0)

    def pair_step(i, carry):
        j0 = 2 * i
        j1 = j0 + 1
        fire(j1, b1, sem1)
        drain(j0, b0, sem0)
        compute(j0, b0)

        @pl.when(j0 + 2 < N_CH)
        def _():
            fire(j0 + 2, b0, sem0)

        drain(j1, b1, sem1)
        compute(j1, b1)
        return carry

    lax.fori_loop(0, N_CH // 2, pair_step, 0)
    pltpu.sync_copy(sc_out, out_hbm.at[pl.ds(base, N_CH)])


_sc_scores = functools.partial(
    pl.kernel,
    out_type=jax.ShapeDtypeStruct((ROWS, CHUNK), jnp.float32),
    mesh=plsc.VectorSubcoreMesh(core_axis_name="c", subcore_axis_name="s",
                                num_cores=NC, num_subcores=NS),
    scratch_types=[
        pltpu.VMEM((N_CH, CHUNK), jnp.int32),
        pltpu.VMEM((N_CH, CHUNK), jnp.int32),
        pltpu.VMEM((N_CH, CHUNK), jnp.int32),
        pltpu.VMEM((N_CH, CHUNK), jnp.int32),
        pltpu.VMEM((CHUNK, 128), jnp.int32),
        pltpu.VMEM((CHUNK, 128), jnp.int32),
        pltpu.VMEM((CHUNK, 128), jnp.int32),
        pltpu.VMEM((CHUNK, 128), jnp.int32),
        pltpu.VMEM((N_CH, CHUNK), jnp.float32),
        pltpu.SemaphoreType.DMA,
        pltpu.SemaphoreType.DMA,
    ],
    compiler_params=pltpu.CompilerParams(needs_layout_passes=False),
)(_sc_body)


def _tc_loss_body(s_ref, o_ref):
    s = s_ref[...]
    row = lax.broadcasted_iota(jnp.int32, (ROWS, CHUNK), 0)
    z = jnp.where(row < B // CHUNK, s, -s)
    ls = jnp.minimum(z, 0.0) - jnp.log1p(jnp.exp(-jnp.abs(z)))
    o_ref[...] = jnp.reshape(-jnp.sum(ls) * (1.0 / B), (1, 1))


_tc_loss = pl.pallas_call(
    _tc_loss_body,
    out_shape=jax.ShapeDtypeStruct((1, 1), jnp.float32),
)


def kernel(pos_u, pos_v, neg_u, neg_v,
           word_amplitude_w, word_phase_w, word_amplitude_v, word_phase_v):
    u_all = jnp.concatenate(
        [pos_u.astype(jnp.int32), neg_u.reshape(-1).astype(jnp.int32)])
    v_all = jnp.concatenate(
        [pos_v.astype(jnp.int32), neg_v.reshape(-1).astype(jnp.int32)])
    u4 = ((u_all >> 11) * 512 + (u_all & 511)).reshape(ROWS, CHUNK)
    offu = (((u_all >> 9) & 3) * 32).reshape(ROWS, CHUNK)
    v4 = ((v_all >> 11) * 512 + (v_all & 511)).reshape(ROWS, CHUNK)
    offv = (((v_all >> 9) & 3) * 32).reshape(ROWS, CHUNK)
    pku, pkv = _tc_repack(word_amplitude_w.T, word_phase_w.T,
                          word_amplitude_v.T, word_phase_v.T)
    scores = _sc_scores(u4, offu, v4, offv, pku, pkv)
    return _tc_loss(scores)[0, 0]


# speedup vs baseline: 2.8239x; 1.4135x over previous
import functools

import jax
import jax.numpy as jnp
from jax import lax
from jax.experimental import pallas as pl
from jax.experimental.pallas import tpu as pltpu
from jax.experimental.pallas import tpu_sc as plsc

B = 16384
K = 5
D = 32
N = B * (1 + K)
NC = 2
NS = 16
NW = NC * NS
CHUNK = 128
N_CH = N // (NW * CHUNK)
ROWS = N // CHUNK

V = 1000000
TBLK = 4096
TGRID = (V + TBLK - 1) // TBLK
PACKED_ROWS = TGRID * (TBLK // 4)

TWO_PI = 6.283185307179586
_C0 = 0.9999994436787928
_C1 = -0.49999558165605595
_C2 = 0.04166103279014615
_C3 = -0.0013862747315839738
_C4 = 2.4253192495694853e-05
_C5 = -2.2193949944515623e-07


def _bf16_rne_bits(x):
    u = lax.bitcast_convert_type(x, jnp.int32)
    return u + jnp.int32(0x7FFF) + (lax.shift_right_logical(u, 16) & 1)


def _tr_body(aw_ref, pw_ref, av_ref, pv_ref, ou_ref, ov_ref):
    for aref, pref, oref in ((aw_ref, pw_ref, ou_ref),
                             (av_ref, pv_ref, ov_ref)):
        ra = _bf16_rne_bits(aref[...])
        rp = _bf16_rne_bits(pref[...])
        packed = (rp & jnp.int32(-65536)) | lax.shift_right_logical(ra, 16)
        pieces = [jnp.swapaxes(packed[:, 512 * c:512 * (c + 1)], 0, 1)
                  for c in range(TBLK // 512)]
        oref[...] = jnp.concatenate(
            [jnp.concatenate(pieces[4 * s:4 * s + 4], axis=1)
             for s in range(TBLK // 2048)], axis=0)


_tc_repack = pl.pallas_call(
    _tr_body,
    grid=(TGRID,),
    in_specs=[pl.BlockSpec((D, TBLK), lambda g: (0, g)) for _ in range(4)],
    out_specs=[pl.BlockSpec((TBLK // 4, 128), lambda g: (g, 0)) for _ in range(2)],
    out_shape=[jax.ShapeDtypeStruct((PACKED_ROWS, 128), jnp.int32)
               for _ in range(2)],
)


def _sc_body(u_hbm, ou_hbm, v_hbm, ov_hbm, pku_hbm, pkv_hbm,
             out_hbm, idx_u, off_u, idx_v, off_v,
             bu0, bv0, bu1, bv1, sc_out, sem0, sem1):
    wid = lax.axis_index("s") * NC + lax.axis_index("c")
    base = wid * N_CH
    pltpu.sync_copy(u_hbm.at[pl.ds(base, N_CH)], idx_u)
    pltpu.sync_copy(ou_hbm.at[pl.ds(base, N_CH)], off_u)
    pltpu.sync_copy(v_hbm.at[pl.ds(base, N_CH)], idx_v)
    pltpu.sync_copy(ov_hbm.at[pl.ds(base, N_CH)], off_v)

    def dmas(j, bufs, sem):
        bu, bv = bufs
        return (pltpu.make_async_copy(pku_hbm.at[idx_u.at[j]], bu, sem),
                pltpu.make_async_copy(pkv_hbm.at[idx_v.at[j]], bv, sem))

    def fire(j, bufs, sem):
        for c in dmas(j, bufs, sem):
            c.start()

    def drain(j, bufs, sem):
        for c in dmas(j, bufs, sem):
            c.wait()

    def compute(j, bufs):
        bu, bv = bufs

        def grp(g, carry2):
            rows = g * 16 + lax.iota(jnp.int32, 16)
            ou = off_u[j, pl.ds(g * 16, 16)]
            ov = off_v[j, pl.ds(g * 16, 16)]

            def dim_step(q, acc):
                for k in range(8):
                    dd = q * 8 + k
                    xu = plsc.load_gather(bu, [rows, ou + dd])
                    xv = plsc.load_gather(bv, [rows, ov + dd])
                    a1, p1 = plsc.unpack(plsc.bitcast(xu, jnp.bfloat16),
                                         format=plsc.PackFormat.INTERLEAVED)
                    a2, p2 = plsc.unpack(plsc.bitcast(xv, jnp.bfloat16),
                                         format=plsc.PackFormat.INTERLEAVED)
                    ax = jnp.abs(p1 - p2)
                    y = jnp.minimum(ax, TWO_PI - ax)
                    uu = y * y
                    cosv = _C0 + uu * (_C1 + uu * (_C2 + uu * (
                        _C3 + uu * (_C4 + uu * _C5))))
                    acc = acc + a1 * a2 * cosv
                return acc

            acc = lax.fori_loop(0, D // 8, dim_step,
                                jnp.zeros((16,), jnp.float32))
            sc_out[j, pl.ds(g * 16, 16)] = acc
            return carry2

        lax.fori_loop(0, CHUNK // 16, grp, 0)

    b0 = (bu0, bv0)
    b1 = (bu1, bv1)
    fire(0, b0, sem0)

    def pair_step(i, carry):
        j0 = 2 * i
        j1 = j0 + 1
        fire(j1, b1, sem1)
        drain(j0, b0, sem0)
        compute(j0, b0)

        @pl.when(j0 + 2 < N_CH)
        def _():
            fire(j0 + 2, b0, sem0)

        drain(j1, b1, sem1)
        compute(j1, b1)
        return carry

    lax.fori_loop(0, N_CH // 2, pair_step, 0)
    pltpu.sync_copy(sc_out, out_hbm.at[pl.ds(base, N_CH)])


_sc_scores = functools.partial(
    pl.kernel,
    out_type=jax.ShapeDtypeStruct((ROWS, CHUNK), jnp.float32),
    mesh=plsc.VectorSubcoreMesh(core_axis_name="c", subcore_axis_name="s",
                                num_cores=NC, num_subcores=NS),
    scratch_types=[
        pltpu.VMEM((N_CH, CHUNK), jnp.int32),
        pltpu.VMEM((N_CH, CHUNK), jnp.int32),
        pltpu.VMEM((N_CH, CHUNK), jnp.int32),
        pltpu.VMEM((N_CH, CHUNK), jnp.int32),
        pltpu.VMEM((CHUNK, 128), jnp.int32),
        pltpu.VMEM((CHUNK, 128), jnp.int32),
        pltpu.VMEM((CHUNK, 128), jnp.int32),
        pltpu.VMEM((CHUNK, 128), jnp.int32),
        pltpu.VMEM((N_CH, CHUNK), jnp.float32),
        pltpu.SemaphoreType.DMA,
        pltpu.SemaphoreType.DMA,
    ],
    compiler_params=pltpu.CompilerParams(needs_layout_passes=False),
)(_sc_body)


def _tc_loss_body(s_ref, o_ref):
    s = s_ref[...]
    row = lax.broadcasted_iota(jnp.int32, (ROWS, CHUNK), 0)
    z = jnp.where(row < B // CHUNK, s, -s)
    ls = jnp.minimum(z, 0.0) - jnp.log1p(jnp.exp(-jnp.abs(z)))
    o_ref[...] = jnp.reshape(-jnp.sum(ls) * (1.0 / B), (1, 1))


_tc_loss = pl.pallas_call(
    _tc_loss_body,
    out_shape=jax.ShapeDtypeStruct((1, 1), jnp.float32),
)


def kernel(pos_u, pos_v, neg_u, neg_v,
           word_amplitude_w, word_phase_w, word_amplitude_v, word_phase_v):
    u_all = jnp.concatenate(
        [pos_u.astype(jnp.int32), neg_u.reshape(-1).astype(jnp.int32)])
    v_all = jnp.concatenate(
        [pos_v.astype(jnp.int32), neg_v.reshape(-1).astype(jnp.int32)])
    u4 = ((u_all >> 11) * 512 + (u_all & 511)).reshape(ROWS, CHUNK)
    offu = (((u_all >> 9) & 3) * 32).reshape(ROWS, CHUNK)
    v4 = ((v_all >> 11) * 512 + (v_all & 511)).reshape(ROWS, CHUNK)
    offv = (((v_all >> 9) & 3) * 32).reshape(ROWS, CHUNK)
    pku, pkv = _tc_repack(word_amplitude_w.T, word_phase_w.T,
                          word_amplitude_v.T, word_phase_v.T)
    scores = _sc_scores(u4, offu, v4, offv, pku, pkv)
    return _tc_loss(scores)[0, 0]


# speedup vs baseline: 2.8874x; 1.0225x over previous
import functools

import jax
import jax.numpy as jnp
from jax import lax
from jax.experimental import pallas as pl
from jax.experimental.pallas import tpu as pltpu
from jax.experimental.pallas import tpu_sc as plsc

B = 16384
K = 5
D = 32
N = B * (1 + K)
NC = 2
NS = 16
NW = NC * NS
CHUNK = 128
N_CH = N // (NW * CHUNK)
ROWS = N // CHUNK

V = 1000000
TBLK = 8192
TGRID = (V + TBLK - 1) // TBLK
PACKED_ROWS = TGRID * (TBLK // 4)

TWO_PI = 6.283185307179586
_C0 = 0.9999994436787928
_C1 = -0.49999558165605595
_C2 = 0.04166103279014615
_C3 = -0.0013862747315839738
_C4 = 2.4253192495694853e-05
_C5 = -2.2193949944515623e-07


def _bf16_rne_bits(x):
    u = lax.bitcast_convert_type(x, jnp.int32)
    return u + jnp.int32(0x7FFF) + (lax.shift_right_logical(u, 16) & 1)


def _tr_body(aw_ref, pw_ref, av_ref, pv_ref, ou_ref, ov_ref):
    for aref, pref, oref in ((aw_ref, pw_ref, ou_ref),
                             (av_ref, pv_ref, ov_ref)):
        ra = _bf16_rne_bits(aref[...])
        rp = _bf16_rne_bits(pref[...])
        packed = (rp & jnp.int32(-65536)) | lax.shift_right_logical(ra, 16)
        pieces = [jnp.swapaxes(packed[:, 512 * c:512 * (c + 1)], 0, 1)
                  for c in range(TBLK // 512)]
        oref[...] = jnp.concatenate(
            [jnp.concatenate(pieces[4 * s:4 * s + 4], axis=1)
             for s in range(TBLK // 2048)], axis=0)


_tc_repack = pl.pallas_call(
    _tr_body,
    grid=(TGRID,),
    in_specs=[pl.BlockSpec((D, TBLK), lambda g: (0, g)) for _ in range(4)],
    out_specs=[pl.BlockSpec((TBLK // 4, 128), lambda g: (g, 0)) for _ in range(2)],
    out_shape=[jax.ShapeDtypeStruct((PACKED_ROWS, 128), jnp.int32)
               for _ in range(2)],
)


def _sc_body(u_hbm, ou_hbm, v_hbm, ov_hbm, pku_hbm, pkv_hbm,
             out_hbm, idx_u, off_u, idx_v, off_v,
             bu0, bv0, bu1, bv1, sc_out, sem0, sem1):
    wid = lax.axis_index("s") * NC + lax.axis_index("c")
    base = wid * N_CH
    pltpu.sync_copy(u_hbm.at[pl.ds(base, N_CH)], idx_u)
    pltpu.sync_copy(ou_hbm.at[pl.ds(base, N_CH)], off_u)
    pltpu.sync_copy(v_hbm.at[pl.ds(base, N_CH)], idx_v)
    pltpu.sync_copy(ov_hbm.at[pl.ds(base, N_CH)], off_v)

    def dmas(j, bufs, sem):
        bu, bv = bufs
        return (pltpu.make_async_copy(pku_hbm.at[idx_u.at[j]], bu, sem),
                pltpu.make_async_copy(pkv_hbm.at[idx_v.at[j]], bv, sem))

    def fire(j, bufs, sem):
        for c in dmas(j, bufs, sem):
            c.start()

    def drain(j, bufs, sem):
        for c in dmas(j, bufs, sem):
            c.wait()

    def compute(j, bufs):
        bu, bv = bufs

        def grp(g, carry2):
            rows = g * 16 + lax.iota(jnp.int32, 16)
            ou = off_u[j, pl.ds(g * 16, 16)]
            ov = off_v[j, pl.ds(g * 16, 16)]

            def dim_step(q, acc):
                for k in range(8):
                    dd = q * 8 + k
                    xu = plsc.load_gather(bu, [rows, ou + dd])
                    xv = plsc.load_gather(bv, [rows, ov + dd])
                    a1, p1 = plsc.unpack(plsc.bitcast(xu, jnp.bfloat16),
                                         format=plsc.PackFormat.INTERLEAVED)
                    a2, p2 = plsc.unpack(plsc.bitcast(xv, jnp.bfloat16),
                                         format=plsc.PackFormat.INTERLEAVED)
                    ax = jnp.abs(p1 - p2)
                    y = jnp.minimum(ax, TWO_PI - ax)
                    uu = y * y
                    cosv = _C0 + uu * (_C1 + uu * (_C2 + uu * (
                        _C3 + uu * (_C4 + uu * _C5))))
                    acc = acc + a1 * a2 * cosv
                return acc

            acc = lax.fori_loop(0, D // 8, dim_step,
                                jnp.zeros((16,), jnp.float32))
            sc_out[j, pl.ds(g * 16, 16)] = acc
            return carry2

        lax.fori_loop(0, CHUNK // 16, grp, 0)

    b0 = (bu0, bv0)
    b1 = (bu1, bv1)
    fire(0, b0, sem0)

    def pair_step(i, carry):
        j0 = 2 * i
        j1 = j0 + 1
        fire(j1, b1, sem1)
        drain(j0, b0, sem0)
        compute(j0, b0)

        @pl.when(j0 + 2 < N_CH)
        def _():
            fire(j0 + 2, b0, sem0)

        drain(j1, b1, sem1)
        compute(j1, b1)
        return carry

    lax.fori_loop(0, N_CH // 2, pair_step, 0)
    pltpu.sync_copy(sc_out, out_hbm.at[pl.ds(base, N_CH)])


_sc_scores = functools.partial(
    pl.kernel,
    out_type=jax.ShapeDtypeStruct((ROWS, CHUNK), jnp.float32),
    mesh=plsc.VectorSubcoreMesh(core_axis_name="c", subcore_axis_name="s",
                                num_cores=NC, num_subcores=NS),
    scratch_types=[
        pltpu.VMEM((N_CH, CHUNK), jnp.int32),
        pltpu.VMEM((N_CH, CHUNK), jnp.int32),
        pltpu.VMEM((N_CH, CHUNK), jnp.int32),
        pltpu.VMEM((N_CH, CHUNK), jnp.int32),
        pltpu.VMEM((CHUNK, 128), jnp.int32),
        pltpu.VMEM((CHUNK, 128), jnp.int32),
        pltpu.VMEM((CHUNK, 128), jnp.int32),
        pltpu.VMEM((CHUNK, 128), jnp.int32),
        pltpu.VMEM((N_CH, CHUNK), jnp.float32),
        pltpu.SemaphoreType.DMA,
        pltpu.SemaphoreType.DMA,
    ],
    compiler_params=pltpu.CompilerParams(needs_layout_passes=False),
)(_sc_body)


def _tc_loss_body(s_ref, o_ref):
    s = s_ref[...]
    row = lax.broadcasted_iota(jnp.int32, (ROWS, CHUNK), 0)
    z = jnp.where(row < B // CHUNK, s, -s)
    ls = jnp.minimum(z, 0.0) - jnp.log1p(jnp.exp(-jnp.abs(z)))
    o_ref[...] = jnp.reshape(-jnp.sum(ls) * (1.0 / B), (1, 1))


_tc_loss = pl.pallas_call(
    _tc_loss_body,
    out_shape=jax.ShapeDtypeStruct((1, 1), jnp.float32),
)


def kernel(pos_u, pos_v, neg_u, neg_v,
           word_amplitude_w, word_phase_w, word_amplitude_v, word_phase_v):
    u_all = jnp.concatenate(
        [pos_u.astype(jnp.int32), neg_u.reshape(-1).astype(jnp.int32)])
    v_all = jnp.concatenate(
        [pos_v.astype(jnp.int32), neg_v.reshape(-1).astype(jnp.int32)])
    u4 = ((u_all >> 11) * 512 + (u_all & 511)).reshape(ROWS, CHUNK)
    offu = (((u_all >> 9) & 3) * 32).reshape(ROWS, CHUNK)
    v4 = ((v_all >> 11) * 512 + (v_all & 511)).reshape(ROWS, CHUNK)
    offv = (((v_all >> 9) & 3) * 32).reshape(ROWS, CHUNK)
    pku, pkv = _tc_repack(word_amplitude_w.T, word_phase_w.T,
                          word_amplitude_v.T, word_phase_v.T)
    scores = _sc_scores(u4, offu, v4, offv, pku, pkv)
    return _tc_loss(scores)[0, 0]
